# baseline (speedup 1.0000x reference)
import functools

import jax
import jax.numpy as jnp
from jax import lax
from jax.experimental import pallas as pl
from jax.experimental.pallas import tpu as pltpu
from jax.experimental.pallas import tpu_sc as plsc

NC = 2
NS = 16
L = 16
NW = NC * NS

WD_I = 0.01
WD_U = 0.01


def _sc_body(dim, b_per_w, n_chunks,
             u_idx, p_idx, n_idx, uf, itf,
             d_out, norms_out,
             idx_u_v, idx_p_v, idx_n_v,
             u_rows, p_rows, n_rows,
             d_v, norms_v, sem):
    wid = lax.axis_index("s") * NC + lax.axis_index("c")
    base = wid * b_per_w

    pltpu.sync_copy(u_idx.at[wid], idx_u_v)
    pltpu.sync_copy(p_idx.at[wid], idx_p_v)
    pltpu.sync_copy(n_idx.at[wid], idx_n_v)

    copies = []
    for j in range(n_chunks):
        sl = pl.ds(j * 128, 128)
        copies.append(pltpu.async_copy(uf.at[idx_u_v.at[j]], u_rows.at[sl], sem))
        copies.append(pltpu.async_copy(itf.at[idx_p_v.at[j]], p_rows.at[sl], sem))
        copies.append(pltpu.async_copy(itf.at[idx_n_v.at[j]], n_rows.at[sl], sem))
    for c in copies:
        c.wait()

    iota = lax.iota(jnp.int32, L)
    zeros = jnp.zeros((L,), jnp.float32)

    def group(g, carry):
        row_ids = g * L + iota
        accd = zeros
        accu = zeros
        accp = zeros
        accn = zeros
        for dd in range(dim):
            ds = jnp.full((L,), dd, jnp.int32)
            gu = plsc.load_gather(u_rows, [row_ids, ds])
            gp = plsc.load_gather(p_rows, [row_ids, ds])
            gn = plsc.load_gather(n_rows, [row_ids, ds])
            accd = accd + gu * (gp - gn)
            accu = accu + gu * gu
            accp = accp + gp * gp
            accn = accn + gn * gn
        out_sl = pl.ds(g * L, L)
        d_v[out_sl] = accd
        norms_v[out_sl] = (accp + accn) * WD_I + accu * WD_U
        return carry

    lax.fori_loop(0, b_per_w // L, group, None)

    pltpu.sync_copy(d_v, d_out.at[pl.ds(base, b_per_w)])
    pltpu.sync_copy(norms_v, norms_out.at[pl.ds(base, b_per_w)])


def _tc_body(inv_b, d_ref, norms_ref, bpr_ref, mean_ref):
    x = d_ref[...]
    bpr = jnp.maximum(-x, 0.0) + jnp.log1p(jnp.exp(-jnp.abs(x)))
    bpr_ref[...] = bpr
    mean_ref[0, 0] = jnp.sum(bpr + norms_ref[...]) * inv_b


def kernel(u_batch, i_batch_pos, i_batch_neg, user_factors, item_factors):
    b = u_batch.shape[0]
    dim = user_factors.shape[1]
    b_per_w = b // NW
    n_chunks = b_per_w // 128

    ui = u_batch.astype(jnp.int32).reshape(NW, n_chunks, 128)
    pi = i_batch_pos.astype(jnp.int32).reshape(NW, n_chunks, 128)
    ni = i_batch_neg.astype(jnp.int32).reshape(NW, n_chunks, 128)

    mesh = plsc.VectorSubcoreMesh(core_axis_name="c", subcore_axis_name="s")
    sc = pl.kernel(
        functools.partial(_sc_body, dim, b_per_w, n_chunks),
        out_type=[
            jax.ShapeDtypeStruct((b,), jnp.float32),
            jax.ShapeDtypeStruct((b,), jnp.float32),
        ],
        mesh=mesh,
        compiler_params=pltpu.CompilerParams(
            needs_layout_passes=False, use_tc_tiling_on_sc=False),
        scratch_types=[
            pltpu.VMEM((n_chunks, 128), jnp.int32),
            pltpu.VMEM((n_chunks, 128), jnp.int32),
            pltpu.VMEM((n_chunks, 128), jnp.int32),
            pltpu.VMEM((b_per_w, dim), jnp.float32),
            pltpu.VMEM((b_per_w, dim), jnp.float32),
            pltpu.VMEM((b_per_w, dim), jnp.float32),
            pltpu.VMEM((b_per_w,), jnp.float32),
            pltpu.VMEM((b_per_w,), jnp.float32),
            pltpu.SemaphoreType.DMA,
        ],
    )
    d, norms = sc(ui, pi, ni, user_factors, item_factors)

    rows = b // 128
    bpr2d, mean = pl.pallas_call(
        functools.partial(_tc_body, 1.0 / b),
        out_shape=[
            jax.ShapeDtypeStruct((rows, 128), jnp.float32),
            jax.ShapeDtypeStruct((1, 1), jnp.float32),
        ],
        out_specs=[
            pl.BlockSpec(memory_space=pltpu.VMEM),
            pl.BlockSpec(memory_space=pltpu.SMEM),
        ],
    )(d.reshape(rows, 128), norms.reshape(rows, 128))

    return (mean[0, 0], bpr2d.reshape(b))
